# baseline (speedup 1.0000x reference)
import functools

import jax
import jax.numpy as jnp
from jax.experimental import pallas as pl
from jax.experimental.pallas import tpu as pltpu

_ROWS = 64
_COLS = 8192
_BLK = 1024
_GRID = _COLS // _BLK
_INF = float("inf")


def _tc_body(xl_ref, xu_ref, outl_ref, outu_ref, cka, ckb, cxl, cxu):
    i = pl.program_id(0)
    xl = xl_ref[...]
    xu = xu_ref[...]
    ka = jnp.float32(0.7) * xl + jnp.float32(0.3) * xu
    kb = jnp.float32(0.3) * xl + jnp.float32(0.7) * xu
    minka = jnp.min(ka, axis=1, keepdims=True)
    kbm = jnp.where(ka == minka, kb, _INF)
    minkb = jnp.min(kbm, axis=1, keepdims=True)
    sel = kbm == minkb
    xlw = jnp.min(jnp.where(sel, xl, _INF), axis=1, keepdims=True)
    xuw = jnp.min(jnp.where(sel, xu, _INF), axis=1, keepdims=True)

    @pl.when(i == 0)
    def _():
        cka[...] = minka
        ckb[...] = minkb
        cxl[...] = xlw
        cxu[...] = xuw

    @pl.when(i > 0)
    def _():
        bka = cka[...]
        bkb = ckb[...]
        better = (minka < bka) | ((minka == bka) & (minkb < bkb))
        cka[...] = jnp.where(better, minka, bka)
        ckb[...] = jnp.where(better, minkb, bkb)
        cxl[...] = jnp.where(better, xlw, cxl[...])
        cxu[...] = jnp.where(better, xuw, cxu[...])

    @pl.when(i == _GRID - 1)
    def _():
        outl_ref[...] = cxl[...]
        outu_ref[...] = cxu[...]


@jax.jit
def kernel(xl, xu):
    return pl.pallas_call(
        _tc_body,
        grid=(_GRID,),
        in_specs=[
            pl.BlockSpec((_ROWS, _BLK), lambda i: (0, i)),
            pl.BlockSpec((_ROWS, _BLK), lambda i: (0, i)),
        ],
        out_specs=(
            pl.BlockSpec((_ROWS, 1), lambda i: (0, 0)),
            pl.BlockSpec((_ROWS, 1), lambda i: (0, 0)),
        ),
        out_shape=(
            jax.ShapeDtypeStruct((_ROWS, 1), jnp.float32),
            jax.ShapeDtypeStruct((_ROWS, 1), jnp.float32),
        ),
        scratch_shapes=[
            pltpu.VMEM((_ROWS, 1), jnp.float32),
            pltpu.VMEM((_ROWS, 1), jnp.float32),
            pltpu.VMEM((_ROWS, 1), jnp.float32),
            pltpu.VMEM((_ROWS, 1), jnp.float32),
        ],
    )(xl, xu)


# speedup vs baseline: 1.0175x; 1.0175x over previous
import jax
import jax.numpy as jnp
from jax.experimental import pallas as pl
from jax.experimental.pallas import tpu as pltpu

_ROWS = 64
_COLS = 8192
_BLK = 1024
_GRID = _COLS // _BLK
_INF = float("inf")


def _tc_body(xl_ref, xu_ref, outl_ref, outu_ref, cka, ckb, cxl, cxu):
    i = pl.program_id(0)
    xl = xl_ref[...]
    xu = xu_ref[...]
    ka = jnp.float32(0.7) * xl + jnp.float32(0.3) * xu
    kb = jnp.float32(0.3) * xl + jnp.float32(0.7) * xu

    @pl.when(i == 0)
    def _():
        cka[...] = ka
        ckb[...] = kb
        cxl[...] = xl
        cxu[...] = xu

    @pl.when(i > 0)
    def _():
        bka = cka[...]
        bkb = ckb[...]
        better = (ka < bka) | ((ka == bka) & (kb < bkb))
        cka[...] = jnp.where(better, ka, bka)
        ckb[...] = jnp.where(better, kb, bkb)
        cxl[...] = jnp.where(better, xl, cxl[...])
        cxu[...] = jnp.where(better, xu, cxu[...])

    @pl.when(i == _GRID - 1)
    def _():
        fka = cka[...]
        fkb = ckb[...]
        minka = jnp.min(fka, axis=1, keepdims=True)
        kbm = jnp.where(fka == minka, fkb, _INF)
        minkb = jnp.min(kbm, axis=1, keepdims=True)
        sel = kbm == minkb
        outl_ref[...] = jnp.min(jnp.where(sel, cxl[...], _INF), axis=1,
                                keepdims=True)
        outu_ref[...] = jnp.min(jnp.where(sel, cxu[...], _INF), axis=1,
                                keepdims=True)


@jax.jit
def kernel(xl, xu):
    return pl.pallas_call(
        _tc_body,
        grid=(_GRID,),
        in_specs=[
            pl.BlockSpec((_ROWS, _BLK), lambda i: (0, i)),
            pl.BlockSpec((_ROWS, _BLK), lambda i: (0, i)),
        ],
        out_specs=(
            pl.BlockSpec((_ROWS, 1), lambda i: (0, 0)),
            pl.BlockSpec((_ROWS, 1), lambda i: (0, 0)),
        ),
        out_shape=(
            jax.ShapeDtypeStruct((_ROWS, 1), jnp.float32),
            jax.ShapeDtypeStruct((_ROWS, 1), jnp.float32),
        ),
        scratch_shapes=[
            pltpu.VMEM((_ROWS, _BLK), jnp.float32),
            pltpu.VMEM((_ROWS, _BLK), jnp.float32),
            pltpu.VMEM((_ROWS, _BLK), jnp.float32),
            pltpu.VMEM((_ROWS, _BLK), jnp.float32),
        ],
    )(xl, xu)


# speedup vs baseline: 1.2107x; 1.1899x over previous
import jax
import jax.numpy as jnp
from jax.experimental import pallas as pl
from jax.experimental.pallas import tpu as pltpu

_ROWS = 64
_COLS = 8192
_BLK = 4096
_GRID = _COLS // _BLK
_INF = float("inf")


def _tc_body(xl_ref, xu_ref, outl_ref, outu_ref, cka, ckb, cxl, cxu):
    i = pl.program_id(0)
    xl = xl_ref[...]
    xu = xu_ref[...]
    ka = jnp.float32(0.7) * xl + jnp.float32(0.3) * xu
    kb = jnp.float32(0.3) * xl + jnp.float32(0.7) * xu

    @pl.when(i == 0)
    def _():
        cka[...] = ka
        ckb[...] = kb
        cxl[...] = xl
        cxu[...] = xu

    @pl.when(i > 0)
    def _():
        bka = cka[...]
        bkb = ckb[...]
        better = (ka < bka) | ((ka == bka) & (kb < bkb))
        cka[...] = jnp.where(better, ka, bka)
        ckb[...] = jnp.where(better, kb, bkb)
        cxl[...] = jnp.where(better, xl, cxl[...])
        cxu[...] = jnp.where(better, xu, cxu[...])

    @pl.when(i == _GRID - 1)
    def _():
        fka = cka[...]
        fkb = ckb[...]
        minka = jnp.min(fka, axis=1, keepdims=True)
        kbm = jnp.where(fka == minka, fkb, _INF)
        minkb = jnp.min(kbm, axis=1, keepdims=True)
        sel = kbm == minkb
        outl_ref[...] = jnp.min(jnp.where(sel, cxl[...], _INF), axis=1,
                                keepdims=True)
        outu_ref[...] = jnp.min(jnp.where(sel, cxu[...], _INF), axis=1,
                                keepdims=True)


@jax.jit
def kernel(xl, xu):
    return pl.pallas_call(
        _tc_body,
        grid=(_GRID,),
        in_specs=[
            pl.BlockSpec((_ROWS, _BLK), lambda i: (0, i)),
            pl.BlockSpec((_ROWS, _BLK), lambda i: (0, i)),
        ],
        out_specs=(
            pl.BlockSpec((_ROWS, 1), lambda i: (0, 0)),
            pl.BlockSpec((_ROWS, 1), lambda i: (0, 0)),
        ),
        out_shape=(
            jax.ShapeDtypeStruct((_ROWS, 1), jnp.float32),
            jax.ShapeDtypeStruct((_ROWS, 1), jnp.float32),
        ),
        scratch_shapes=[
            pltpu.VMEM((_ROWS, _BLK), jnp.float32),
            pltpu.VMEM((_ROWS, _BLK), jnp.float32),
            pltpu.VMEM((_ROWS, _BLK), jnp.float32),
            pltpu.VMEM((_ROWS, _BLK), jnp.float32),
        ],
    )(xl, xu)
